# baseline (speedup 1.0000x reference)
import functools

import jax
import jax.numpy as jnp
from jax import lax
from jax.experimental import pallas as pl
from jax.experimental.pallas import tpu as pltpu
from jax.experimental.pallas import tpu_sc as plsc

N = 10000
D = 128
S = 32
OUT = 128

NW = 32
NPW = 320
N_PAD = NW * NPW
G = 4
CHUNKS = NPW // G


def _sc_maxpool(h, idx_flat):
    mesh = plsc.VectorSubcoreMesh(core_axis_name="c", subcore_axis_name="s")

    @functools.partial(
        pl.kernel,
        mesh=mesh,
        out_type=jax.ShapeDtypeStruct((N_PAD, D), jnp.float32),
        scratch_types=[
            pltpu.VMEM((G * S,), jnp.int32),
            pltpu.VMEM((G * S, D), jnp.float32),
            pltpu.VMEM((G, D), jnp.float32),
            pltpu.SemaphoreType.DMA,
        ],
    )
    def pool(h_hbm, idx_hbm, out_hbm, idx_v, rows_v, pool_v, sem):
        wid = lax.axis_index("s") * 2 + lax.axis_index("c")
        base = wid * NPW

        def chunk(ci, carry):
            nb = base + ci * G
            pltpu.sync_copy(idx_hbm.at[pl.ds(nb * S, G * S)], idx_v)
            pltpu.async_copy(h_hbm.at[idx_v], rows_v, sem).wait()
            for g in range(G):
                for c in range(D // 16):
                    sl = pl.ds(c * 16, 16)
                    acc = rows_v[g * S, sl]
                    for t in range(1, S):
                        acc = jnp.maximum(acc, rows_v[g * S + t, sl])
                    pool_v[g, sl] = acc
            pltpu.sync_copy(pool_v, out_hbm.at[pl.ds(nb, G)])
            return carry

        lax.fori_loop(0, CHUNKS, chunk, 0)

    return pool(h, idx_flat)


_BR = 400


def _tc_matmul(h, pooled, W):
    def body(h_ref, p_ref, w_ref, o_ref):
        o_ref[...] = jnp.dot(
            h_ref[...], w_ref[0:D, :], preferred_element_type=jnp.float32
        ) + jnp.dot(
            p_ref[...], w_ref[D : 2 * D, :], preferred_element_type=jnp.float32
        )

    return pl.pallas_call(
        body,
        grid=(N // _BR,),
        in_specs=[
            pl.BlockSpec((_BR, D), lambda i: (i, 0)),
            pl.BlockSpec((_BR, D), lambda i: (i, 0)),
            pl.BlockSpec((2 * D, OUT), lambda i: (0, 0)),
        ],
        out_specs=pl.BlockSpec((_BR, OUT), lambda i: (i, 0)),
        out_shape=jax.ShapeDtypeStruct((N, OUT), jnp.float32),
    )(h, pooled, W)


def kernel(h, adj_list, aggregate_num, aggregate_neighbors, W):
    idx = jnp.pad(aggregate_neighbors, ((0, N_PAD - N), (0, 0)))
    pooled = _sc_maxpool(h, idx.reshape(-1))
    return _tc_matmul(h, pooled[:N], W)


# speedup vs baseline: 1.3414x; 1.3414x over previous
import functools

import jax
import jax.numpy as jnp
from jax import lax
from jax.experimental import pallas as pl
from jax.experimental.pallas import tpu as pltpu
from jax.experimental.pallas import tpu_sc as plsc

N = 10000
D = 128
S = 32
OUT = 128

NW = 32
NPW = 320
N_PAD = NW * NPW
G = 4
CHUNKS = NPW // G


def _sc_maxpool(h, idx3d):
    mesh = plsc.VectorSubcoreMesh(core_axis_name="c", subcore_axis_name="s")

    @functools.partial(
        pl.kernel,
        mesh=mesh,
        out_type=jax.ShapeDtypeStruct((N_PAD, D), jnp.float32),
        scratch_types=[
            pltpu.VMEM((CHUNKS, G * S), jnp.int32),
            pltpu.VMEM((G * S, D), jnp.float32),
            pltpu.VMEM((G * S, D), jnp.float32),
            pltpu.VMEM((NPW, D), jnp.float32),
            pltpu.SemaphoreType.DMA,
            pltpu.SemaphoreType.DMA,
        ],
    )
    def pool(h_hbm, idx_hbm, out_hbm, idx_v, rows0, rows1, pool_v, sem0, sem1):
        wid = lax.axis_index("s") * 2 + lax.axis_index("c")
        base = wid * NPW
        rows = (rows0, rows1)
        sems = (sem0, sem1)

        pltpu.sync_copy(idx_hbm.at[wid], idx_v)

        def gather(ci, b):
            return pltpu.make_async_copy(h_hbm.at[idx_v.at[ci]], rows[b], sems[b])

        gather(0, 0).start()
        gather(1, 1).start()

        def body(i, carry):
            for b in range(2):
                ci = i * 2 + b
                gather(ci, b).wait()
                for g in range(G):
                    row_out = ci * G + g
                    for c in range(D // 16):
                        sl = pl.ds(c * 16, 16)
                        acc = rows[b][g * S, sl]
                        for t in range(1, S):
                            acc = jnp.maximum(acc, rows[b][g * S + t, sl])
                        pool_v[row_out, sl] = acc
                nxt = ci + 2

                @pl.when(nxt < CHUNKS)
                def _():
                    gather(nxt, b).start()

            return carry

        lax.fori_loop(0, CHUNKS // 2, body, 0)
        pltpu.sync_copy(pool_v, out_hbm.at[pl.ds(base, NPW)])

    return pool(h, idx3d)


_BR = 400


def _tc_matmul(h, pooled, W):
    def body(h_ref, p_ref, w_ref, o_ref):
        o_ref[...] = jnp.dot(
            h_ref[...], w_ref[0:D, :], preferred_element_type=jnp.float32
        ) + jnp.dot(
            p_ref[...], w_ref[D : 2 * D, :], preferred_element_type=jnp.float32
        )

    return pl.pallas_call(
        body,
        grid=(N // _BR,),
        in_specs=[
            pl.BlockSpec((_BR, D), lambda i: (i, 0)),
            pl.BlockSpec((_BR, D), lambda i: (i, 0)),
            pl.BlockSpec((2 * D, OUT), lambda i: (0, 0)),
        ],
        out_specs=pl.BlockSpec((_BR, OUT), lambda i: (i, 0)),
        out_shape=jax.ShapeDtypeStruct((N, OUT), jnp.float32),
    )(h, pooled, W)


def kernel(h, adj_list, aggregate_num, aggregate_neighbors, W):
    idx = jnp.pad(aggregate_neighbors, ((0, N_PAD - N), (0, 0)))
    pooled = _sc_maxpool(h, idx.reshape(NW, CHUNKS, G * S))
    return _tc_matmul(h, pooled[:N], W)


# speedup vs baseline: 1.6319x; 1.2166x over previous
import functools

import jax
import jax.numpy as jnp
from jax import lax
from jax.experimental import pallas as pl
from jax.experimental.pallas import tpu as pltpu
from jax.experimental.pallas import tpu_sc as plsc

N = 10000
D = 128
S = 32
OUT = 128

NW = 32
NPW = 320
N_PAD = NW * NPW
G = 4
CHUNKS = NPW // G

D2 = D // 2


def _sc_maxpool(h_pk, idx3d):
    mesh = plsc.VectorSubcoreMesh(core_axis_name="c", subcore_axis_name="s")

    @functools.partial(
        pl.kernel,
        mesh=mesh,
        compiler_params=pltpu.CompilerParams(use_tc_tiling_on_sc=False),
        out_type=jax.ShapeDtypeStruct((NW, CHUNKS, G, D2), jnp.int32),
        scratch_types=[
            pltpu.VMEM((CHUNKS, G * S), jnp.int32),
            pltpu.VMEM((G * S,), jnp.int32),
            pltpu.VMEM((G * S,), jnp.int32),
            pltpu.VMEM((G * S, D2), jnp.int32),
            pltpu.VMEM((G * S, D2), jnp.int32),
            pltpu.VMEM((CHUNKS, G, D2), jnp.int32),
            pltpu.SemaphoreType.DMA,
            pltpu.SemaphoreType.DMA,
        ],
    )
    def pool(
        h_hbm, idx_hbm, out_hbm,
        idx_v, ib0, ib1, rows0, rows1, pool_v, sem0, sem1,
    ):
        sid = lax.axis_index("s")
        wid = sid * 2 + lax.axis_index("c")
        ibs = (ib0, ib1)
        rows = (rows0, rows1)
        sems = (sem0, sem1)

        pltpu.sync_copy(idx_hbm.at[wid], idx_v)

        def stage_idx(ci, b):
            for q in range(G * S // 16):
                sl = pl.ds(q * 16, 16)
                ibs[b][sl] = idx_v[ci, sl]

        def gather(b):
            return pltpu.make_async_copy(h_hbm.at[ibs[b]], rows[b], sems[b])

        stage_idx(0, 0)
        gather(0).start()
        stage_idx(1, 1)
        gather(1).start()

        sh16 = jnp.full((16,), 16, jnp.int32)
        m16 = jnp.full((16,), 0xFFFF, jnp.int32)

        def body(i, carry):
            for b in range(2):
                ci = i * 2 + b
                gather(b).wait()
                for g in range(G):
                    for c in range(D2 // 16):
                        sl = pl.ds(c * 16, 16)
                        w = rows[b][g * S, sl]
                        lo = lax.bitwise_and(w, m16)
                        hi = lax.shift_right_logical(w, sh16)
                        for t in range(1, S):
                            w = rows[b][g * S + t, sl]
                            lo = jnp.maximum(lo, lax.bitwise_and(w, m16))
                            hi = jnp.maximum(hi, lax.shift_right_logical(w, sh16))
                        pool_v[ci, g, sl] = lax.bitwise_or(lo, lax.shift_left(hi, sh16))
                nxt = ci + 2

                @pl.when(nxt < CHUNKS)
                def _():
                    stage_idx(nxt, b)
                    gather(b).start()

            return carry

        lax.fori_loop(0, CHUNKS // 2, body, 0)
        pltpu.sync_copy(pool_v, out_hbm.at[wid])

    return pool(h_pk, idx3d)


_BR = 400


def _tc_matmul(h, pooled, W):
    def body(h_ref, p_ref, w_ref, o_ref):
        o_ref[...] = jnp.dot(
            h_ref[...], w_ref[0:D, :], preferred_element_type=jnp.float32
        ) + jnp.dot(
            p_ref[...].astype(jnp.float32),
            w_ref[D : 2 * D, :],
            preferred_element_type=jnp.float32,
        )

    return pl.pallas_call(
        body,
        grid=(N // _BR,),
        in_specs=[
            pl.BlockSpec((_BR, D), lambda i: (i, 0)),
            pl.BlockSpec((_BR, D), lambda i: (i, 0)),
            pl.BlockSpec((2 * D, OUT), lambda i: (0, 0)),
        ],
        out_specs=pl.BlockSpec((_BR, OUT), lambda i: (i, 0)),
        out_shape=jax.ShapeDtypeStruct((N, OUT), jnp.float32),
    )(h, pooled, W)


def kernel(h, adj_list, aggregate_num, aggregate_neighbors, W):
    idx = jnp.pad(aggregate_neighbors, ((0, N_PAD - N), (0, 0)))
    u = lax.bitcast_convert_type(h.astype(jnp.bfloat16), jnp.uint16).astype(
        jnp.int32
    )
    k = jnp.where(u >= 0x8000, u ^ 0xFFFF, u | 0x8000)
    h_pk = k[:, 0::2] | (k[:, 1::2] << 16)
    out_pk = _sc_maxpool(h_pk, idx.reshape(NW, CHUNKS, G * S))
    pk = out_pk.reshape(N_PAD, D2)[:N]
    lo = pk & 0xFFFF
    hi = (pk >> 16) & 0xFFFF
    inv = lambda q: jnp.where(q >= 0x8000, q ^ 0x8000, q ^ 0xFFFF)
    u16 = jnp.stack([inv(lo), inv(hi)], -1).reshape(N, D).astype(jnp.uint16)
    pooled = lax.bitcast_convert_type(u16, jnp.bfloat16)
    return _tc_matmul(h, pooled, W)


# speedup vs baseline: 2.8616x; 1.7535x over previous
import functools

import jax
import jax.numpy as jnp
from jax import lax
from jax.experimental import pallas as pl
from jax.experimental.pallas import tpu as pltpu
from jax.experimental.pallas import tpu_sc as plsc

N = 10000
D = 128
S = 32
OUT = 128

NW = 32
NPW = 320
N_PAD = NW * NPW
G = 4
CHUNKS = NPW // G

D2 = D // 2


def _sc_maxpool(h_pk, idx3d):
    mesh = plsc.VectorSubcoreMesh(core_axis_name="c", subcore_axis_name="s")

    @functools.partial(
        pl.kernel,
        mesh=mesh,
        compiler_params=pltpu.CompilerParams(use_tc_tiling_on_sc=False),
        out_type=jax.ShapeDtypeStruct((NW, CHUNKS, G, D2), jnp.int32),
        scratch_types=[
            pltpu.VMEM((CHUNKS, G * S), jnp.int32),
            pltpu.VMEM((G * S,), jnp.int32),
            pltpu.VMEM((G * S,), jnp.int32),
            pltpu.VMEM((G * S, D2), jnp.int32),
            pltpu.VMEM((G * S, D2), jnp.int32),
            pltpu.VMEM((CHUNKS, G, D2), jnp.int32),
            pltpu.SemaphoreType.DMA,
            pltpu.SemaphoreType.DMA,
        ],
    )
    def pool(
        h_hbm, idx_hbm, out_hbm,
        idx_v, ib0, ib1, rows0, rows1, pool_v, sem0, sem1,
    ):
        sid = lax.axis_index("s")
        wid = sid * 2 + lax.axis_index("c")
        ibs = (ib0, ib1)
        rows = (rows0, rows1)
        sems = (sem0, sem1)

        pltpu.sync_copy(idx_hbm.at[wid], idx_v)

        def stage_idx(ci, b):
            for q in range(G * S // 16):
                sl = pl.ds(q * 16, 16)
                ibs[b][sl] = idx_v[ci, sl]

        def gather(b):
            return pltpu.make_async_copy(h_hbm.at[ibs[b]], rows[b], sems[b])

        stage_idx(0, 0)
        gather(0).start()
        stage_idx(1, 1)
        gather(1).start()

        sh16 = jnp.full((16,), 16, jnp.int32)
        m16 = jnp.full((16,), 0xFFFF, jnp.int32)

        def body(i, carry):
            for b in range(2):
                ci = i * 2 + b
                gather(b).wait()
                for g in range(G):
                    for c in range(D2 // 16):
                        sl = pl.ds(c * 16, 16)
                        w = rows[b][g * S, sl]
                        lo = lax.bitwise_and(w, m16)
                        hi = lax.shift_right_logical(w, sh16)
                        for t in range(1, S):
                            w = rows[b][g * S + t, sl]
                            lo = jnp.maximum(lo, lax.bitwise_and(w, m16))
                            hi = jnp.maximum(hi, lax.shift_right_logical(w, sh16))
                        pool_v[ci, g, sl] = lax.bitwise_or(lo, lax.shift_left(hi, sh16))
                nxt = ci + 2

                @pl.when(nxt < CHUNKS)
                def _():
                    stage_idx(nxt, b)
                    gather(b).start()

            return carry

        lax.fori_loop(0, CHUNKS // 2, body, 0)
        pltpu.sync_copy(pool_v, out_hbm.at[wid])

    return pool(h_pk, idx3d)


_BR = 400


def _tc_matmul(h, pooled, W):
    def body(h_ref, p_ref, w_ref, o_ref):
        o_ref[...] = jnp.dot(
            h_ref[...], w_ref[0:D, :], preferred_element_type=jnp.float32
        ) + jnp.dot(
            p_ref[...].astype(jnp.float32),
            w_ref[D : 2 * D, :],
            preferred_element_type=jnp.float32,
        )

    return pl.pallas_call(
        body,
        grid=(N // _BR,),
        in_specs=[
            pl.BlockSpec((_BR, D), lambda i: (i, 0)),
            pl.BlockSpec((_BR, D), lambda i: (i, 0)),
            pl.BlockSpec((2 * D, OUT), lambda i: (0, 0)),
        ],
        out_specs=pl.BlockSpec((_BR, OUT), lambda i: (i, 0)),
        out_shape=jax.ShapeDtypeStruct((N, OUT), jnp.float32),
    )(h, pooled, W)


def kernel(h, adj_list, aggregate_num, aggregate_neighbors, W):
    idx = jnp.pad(aggregate_neighbors, ((0, N_PAD - N), (0, 0)))
    u = lax.bitcast_convert_type(h.astype(jnp.bfloat16), jnp.uint16).astype(
        jnp.int32
    )
    k = jnp.where(u >= 0x8000, u ^ 0xFFFF, u | 0x8000)
    h_pk = k[:, :D2] | (k[:, D2:] << 16)
    out_pk = _sc_maxpool(h_pk, idx.reshape(NW, CHUNKS, G * S))
    pk = out_pk.reshape(N_PAD, D2)[:N]
    lo = pk & 0xFFFF
    hi = (pk >> 16) & 0xFFFF
    inv = lambda q: jnp.where(q >= 0x8000, q ^ 0x8000, q ^ 0xFFFF)
    u16 = jnp.concatenate([inv(lo), inv(hi)], axis=1).astype(jnp.uint16)
    pooled = lax.bitcast_convert_type(u16, jnp.bfloat16)
    return _tc_matmul(h, pooled, W)
